# baseline (speedup 1.0000x reference)
import functools

import jax
import jax.numpy as jnp
from jax import lax
from jax.experimental import pallas as pl
from jax.experimental.pallas import tpu as pltpu
from jax.experimental.pallas import tpu_sc as plsc

N = 10000
D = 128
E = 320000
K = 10
ALPHA = 0.1

NC = 2
NS = 16
NW = NC * NS
CH = 40
NJ = E // NW // CH
RPS = N // NS

_BN_INV = 1.0 / (1.0 + 1e-5) ** 0.5
_SQRT_HALF = 0.7071067811865476

_U = 5


def _sc_prop_body(scaled_hbm, row_hbm, col_hbm, zeros_hbm, out_hbm,
                  irow, icol, gb, acc, *sems):
    isems, gsems, ssems = sems[:_U], sems[_U:2 * _U], sems[2 * _U:]
    c = lax.axis_index("c")
    s = lax.axis_index("s")
    wid = c * NS + s

    def load_idx(slot, j):
        pltpu.async_copy(row_hbm.at[wid, j], irow.at[slot], isems[slot])
        pltpu.async_copy(col_hbm.at[wid, j], icol.at[slot], isems[slot])

    def wait_idx(slot, j):
        pltpu.make_async_copy(
            row_hbm.at[wid, j], irow.at[slot], isems[slot]).wait()
        pltpu.make_async_copy(
            col_hbm.at[wid, j], icol.at[slot], isems[slot]).wait()

    def start_gather(slot):
        pltpu.async_copy(scaled_hbm.at[irow.at[slot]], gb.at[slot],
                         gsems[slot])

    def wait_gather(slot):
        pltpu.make_async_copy(scaled_hbm.at[irow.at[slot]], gb.at[slot],
                              gsems[slot]).wait()

    def start_scatter(slot):
        pltpu.async_copy(gb.at[slot], acc.at[icol.at[slot]], ssems[slot],
                         add=True)

    def wait_scatter(slot):
        pltpu.make_async_copy(gb.at[slot], acc.at[icol.at[slot]],
                              ssems[slot]).wait()

    pltpu.sync_copy(zeros_hbm, acc.at[pl.ds(s * RPS, RPS)])

    for p in range(_U):
        load_idx(p, p)
    plsc.subcore_barrier()
    for p in range(2):
        wait_idx(p, p)
        start_gather(p)

    @pl.loop(0, NJ, step=_U)
    def _(j):
        for p in range(_U):
            jj = j + p
            p2 = (p + 2) % _U
            pm = (p - 1) % _U

            @pl.when(jj + 2 < NJ)
            def _():
                wait_idx(p2, jj + 2)
                start_gather(p2)

            wait_gather(p)
            start_scatter(p)

            @pl.when(jj >= 1)
            def _():
                wait_scatter(pm)

                @pl.when(jj - 1 + _U < NJ)
                def _():
                    load_idx(pm, jj - 1 + _U)

    wait_scatter((NJ - 1) % _U)
    plsc.subcore_barrier()
    pltpu.sync_copy(acc.at[pl.ds(s * RPS, RPS)], out_hbm.at[c, s])


NJD = E // NW // CH


def _sc_deg_body(row_hbm, ones_hbm, zeros_hbm, out_hbm, rowbuf, ones_v, dacc,
                 *sems):
    c = lax.axis_index("c")
    s = lax.axis_index("s")
    wid = c * NS + s

    pltpu.sync_copy(row_hbm.at[wid], rowbuf)
    pltpu.sync_copy(ones_hbm, ones_v)
    pltpu.sync_copy(zeros_hbm, dacc.at[pl.ds(s * RPS, RPS)])
    plsc.subcore_barrier()

    def start_sc(slot, j):
        pltpu.async_copy(ones_v, dacc.at[rowbuf.at[j]], sems[slot], add=True)

    def wait_sc(slot):
        pltpu.make_async_copy(ones_v, dacc.at[rowbuf.at[0]],
                              sems[slot]).wait()

    for p in range(_U):
        start_sc(p, p)

    @pl.loop(_U, NJD, step=_U)
    def _(j):
        for p in range(_U):
            wait_sc(p)
            start_sc(p, j + p)

    for p in range(_U):
        wait_sc(p)
    plsc.subcore_barrier()
    pltpu.sync_copy(dacc.at[pl.ds(s * RPS, RPS)], out_hbm.at[c, s])


@functools.lru_cache(maxsize=None)
def _build_sc_kernels():
    mesh = plsc.VectorSubcoreMesh(
        core_axis_name="c", subcore_axis_name="s",
        num_cores=NC, num_subcores=NS)
    prop = pl.kernel(
        _sc_prop_body,
        out_type=jax.ShapeDtypeStruct((NC, NS, RPS, D), jnp.float32),
        mesh=mesh,
        scratch_types=[
            pltpu.VMEM((_U, CH), jnp.int32),
            pltpu.VMEM((_U, CH), jnp.int32),
            pltpu.VMEM((_U, CH, D), jnp.float32),
            pltpu.VMEM_SHARED((N, D), jnp.float32),
        ] + [pltpu.SemaphoreType.DMA] * (3 * _U),
    )
    deg = pl.kernel(
        _sc_deg_body,
        out_type=jax.ShapeDtypeStruct((NC, NS, RPS, D), jnp.float32),
        mesh=mesh,
        scratch_types=[
            pltpu.VMEM((NJD, CH), jnp.int32),
            pltpu.VMEM((CH, D), jnp.float32),
            pltpu.VMEM_SHARED((N, D), jnp.float32),
        ] + [pltpu.SemaphoreType.DMA] * _U,
    )
    return prop, deg


def _sc_prop(scaled, gather_idx, scatter_idx, zeros):
    return _build_sc_kernels()[0](
        scaled, gather_idx, scatter_idx, zeros).reshape(NC, N, D)


def _sc_deg(row, ones, zeros):
    return _build_sc_kernels()[1](row, ones, zeros).reshape(NC, N, D)


_B = 1000


def _mlp_body(x_ref, w1_ref, c1_ref, d1_ref, w2_ref, c2_ref, d2_ref, o_ref):
    x = x_ref[...]
    h = jnp.dot(x, w1_ref[...], preferred_element_type=jnp.float32)
    h = jnp.maximum(h * c1_ref[...] + d1_ref[...], 0.0)
    h2 = jnp.dot(h, w2_ref[...], preferred_element_type=jnp.float32)
    h2 = jnp.maximum(h2 * c2_ref[...] + d2_ref[...], 0.0)
    o_ref[...] = h2 + h


def _ew0_body(d0_ref, d1_ref, base_ref, invd_ref, scaled_ref):
    deg = d0_ref[...] + d1_ref[...]
    iv = 1.0 / jnp.maximum(deg[:, 0:1], 1.0)
    invd_ref[...] = iv
    scaled_ref[...] = base_ref[...] * iv


def _ew_body(p0_ref, p1_ref, base_ref, invd_ref, xs_ref, scaled_ref):
    cur = (1.0 - ALPHA) * (p0_ref[...] + p1_ref[...]) + ALPHA * base_ref[...]
    xs_ref[...] = cur
    scaled_ref[...] = cur * invd_ref[...]


def _final_body(*refs):
    (x0_ref, x1_ref, x2_ref, x3_ref, x4_ref, x5_ref, x6_ref, x7_ref, x8_ref,
     x9_ref, x10_ref, wa1_ref, ba1_ref, wa2_ref, ba2_ref, wh1_ref, dh1_ref,
     ch_ref, wh2_ref, bh2_ref, o_ref) = refs
    xs = [x0_ref[...], x1_ref[...], x2_ref[...], x3_ref[...], x4_ref[...],
          x5_ref[...], x6_ref[...], x7_ref[...], x8_ref[...], x9_ref[...],
          x10_ref[...]]
    ctx = jnp.concatenate([xs[0], xs[K]], axis=1)
    a = jnp.dot(ctx, wa1_ref[...], preferred_element_type=jnp.float32)
    a = a + ba1_ref[...]
    a = 0.5 * a * (1.0 + lax.erf(a * _SQRT_HALF))
    logits = jnp.dot(a, wa2_ref[...], preferred_element_type=jnp.float32)
    logits = logits + ba2_ref[...]
    logits = logits - jnp.max(logits, axis=1, keepdims=True)
    ex = jnp.exp(logits)
    w = ex / jnp.sum(ex, axis=1, keepdims=True)
    fused = w[:, 0:1] * xs[0]
    for k in range(1, K + 1):
        fused = fused + w[:, k:k + 1] * xs[k]
    o1 = jnp.dot(fused, wh1_ref[...], preferred_element_type=jnp.float32)
    o1 = jnp.maximum(o1 * ch_ref[...] + dh1_ref[...], 0.0)
    out = jnp.dot(o1, wh2_ref[...], preferred_element_type=jnp.float32)
    o_ref[...] = out + bh2_ref[...]


def _row_spec(width):
    return pl.BlockSpec((_B, width), lambda i: (i, 0))


def _full_spec(shape):
    return pl.BlockSpec(shape, lambda i: tuple(0 for _ in shape))


def _tc_call(body, in_specs, out_specs, out_shape):
    return pl.pallas_call(
        body,
        grid=(N // _B,),
        in_specs=in_specs,
        out_specs=out_specs,
        out_shape=out_shape,
    )


def kernel(x, edge_index, W1, b1, g1, be1, W2, b2, g2, be2, Wa1, ba1, Wa2,
           ba2, Wh1, bh1, gh, beh, Wh2, bh2):
    f32 = jnp.float32
    row = edge_index[0].reshape(NW, NJ, CH)
    col = edge_index[1].reshape(NW, NJ, CH)
    zeros128 = jnp.zeros((RPS, D), f32)
    ones128 = jnp.ones((CH, D), f32)

    c1 = (g1 * _BN_INV)[None, :]
    d1 = (b1 * g1 * _BN_INV + be1)[None, :]
    c2 = (g2 * _BN_INV)[None, :]
    d2 = (b2 * g2 * _BN_INV + be2)[None, :]
    ch = (gh * _BN_INV)[None, :]
    dh1 = (bh1 * gh * _BN_INV + beh)[None, :]

    base = _tc_call(
        _mlp_body,
        [_row_spec(D), _full_spec((D, D)), _full_spec((1, D)),
         _full_spec((1, D)), _full_spec((D, D)), _full_spec((1, D)),
         _full_spec((1, D))],
        _row_spec(D),
        jax.ShapeDtypeStruct((N, D), f32),
    )(x, W1.T, c1, d1, W2.T, c2, d2)

    degp = _sc_deg(row, ones128, zeros128)

    invd, scaled = _tc_call(
        _ew0_body,
        [_row_spec(D), _row_spec(D), _row_spec(D)],
        [_row_spec(1), _row_spec(D)],
        [jax.ShapeDtypeStruct((N, 1), f32), jax.ShapeDtypeStruct((N, D), f32)],
    )(degp[0], degp[1], base)

    xs = [base]
    for _ in range(K):
        parts = _sc_prop(scaled, row, col, zeros128)
        xt, scaled = _tc_call(
            _ew_body,
            [_row_spec(D), _row_spec(D), _row_spec(D), _row_spec(1)],
            [_row_spec(D), _row_spec(D)],
            [jax.ShapeDtypeStruct((N, D), f32),
             jax.ShapeDtypeStruct((N, D), f32)],
        )(parts[0], parts[1], base, invd)
        xs.append(xt)

    att_hidden = Wa1.shape[0]
    out = _tc_call(
        _final_body,
        [_row_spec(D)] * (K + 1) + [
            _full_spec((2 * D, att_hidden)), _full_spec((1, att_hidden)),
            _full_spec((att_hidden, K + 1)), _full_spec((1, K + 1)),
            _full_spec((D, D // 2)), _full_spec((1, D // 2)),
            _full_spec((1, D // 2)), _full_spec((D // 2, D)),
            _full_spec((1, D)),
        ],
        _row_spec(D),
        jax.ShapeDtypeStruct((N, D), f32),
    )(*xs, Wa1.T, ba1[None, :], Wa2.T, ba2[None, :], Wh1.T, dh1, ch, Wh2.T,
      bh2[None, :])
    return out


# speedup vs baseline: 1.1537x; 1.1537x over previous
import functools

import jax
import jax.numpy as jnp
from jax import lax
from jax.experimental import pallas as pl
from jax.experimental.pallas import tpu as pltpu
from jax.experimental.pallas import tpu_sc as plsc

N = 10000
D = 128
E = 320000
K = 10
ALPHA = 0.1

NC = 2
NS = 16
NW = NC * NS
CH = 40
NJ = E // NW // CH
RPS = N // NS

_BN_INV = 1.0 / (1.0 + 1e-5) ** 0.5
_SQRT_HALF = 0.7071067811865476

_U = 5
_IR = 10
_GD = 4


def _sc_prop_body(scaled_hbm, row_hbm, col_hbm, zeros_hbm, out_hbm,
                  irow, icol, gb, acc, *sems):
    isems, gsems, ssems = sems[:_IR], sems[_IR:_IR + _U], sems[_IR + _U:]
    c = lax.axis_index("c")
    s = lax.axis_index("s")
    wid = c * NS + s

    def load_idx(slot, j):
        pltpu.async_copy(row_hbm.at[wid, j], irow.at[slot], isems[slot])
        pltpu.async_copy(col_hbm.at[wid, j], icol.at[slot], isems[slot])

    def wait_idx(slot, j):
        pltpu.make_async_copy(
            row_hbm.at[wid, j], irow.at[slot], isems[slot]).wait()
        pltpu.make_async_copy(
            col_hbm.at[wid, j], icol.at[slot], isems[slot]).wait()

    def start_gather(gslot, islot):
        pltpu.async_copy(scaled_hbm.at[irow.at[islot]], gb.at[gslot],
                         gsems[gslot])

    def wait_gather(gslot, islot):
        pltpu.make_async_copy(scaled_hbm.at[irow.at[islot]], gb.at[gslot],
                              gsems[gslot]).wait()

    def start_scatter(gslot, islot):
        pltpu.async_copy(gb.at[gslot], acc.at[icol.at[islot]], ssems[gslot],
                         add=True)

    def wait_scatter(gslot, islot):
        pltpu.make_async_copy(gb.at[gslot], acc.at[icol.at[islot]],
                              ssems[gslot]).wait()

    pltpu.sync_copy(zeros_hbm, acc.at[pl.ds(s * RPS, RPS)])

    for p in range(_IR):
        load_idx(p, p)
    plsc.subcore_barrier()
    for p in range(_GD):
        wait_idx(p, p)
        start_gather(p, p)

    @pl.loop(0, NJ, step=_IR)
    def _(j):
        for p in range(_IR):
            jj = j + p
            q = p % _U
            qm, pm = (q - 1) % _U, (p - 1) % _IR
            qg, pg = (q + _GD) % _U, (p + _GD) % _IR

            @pl.when(jj >= 1)
            def _():
                wait_scatter(qm, pm)

                @pl.when(jj - 1 + _IR < NJ)
                def _():
                    load_idx(pm, jj - 1 + _IR)

            @pl.when(jj + _GD < NJ)
            def _():
                wait_idx(pg, jj + _GD)
                start_gather(qg, pg)

            wait_gather(q, p)
            start_scatter(q, p)

    wait_scatter((NJ - 1) % _U, (NJ - 1) % _IR)
    plsc.subcore_barrier()
    pltpu.sync_copy(acc.at[pl.ds(s * RPS, RPS)], out_hbm.at[c, s])


NJD = E // NW // CH


def _sc_deg_body(row_hbm, ones_hbm, zeros_hbm, out_hbm, rowbuf, ones_v, dacc,
                 *sems):
    c = lax.axis_index("c")
    s = lax.axis_index("s")
    wid = c * NS + s

    pltpu.sync_copy(row_hbm.at[wid], rowbuf)
    pltpu.sync_copy(ones_hbm, ones_v)
    pltpu.sync_copy(zeros_hbm, dacc.at[pl.ds(s * RPS, RPS)])
    plsc.subcore_barrier()

    def start_sc(slot, j):
        pltpu.async_copy(ones_v, dacc.at[rowbuf.at[j]], sems[slot], add=True)

    def wait_sc(slot):
        pltpu.make_async_copy(ones_v, dacc.at[rowbuf.at[0]],
                              sems[slot]).wait()

    for p in range(_U):
        start_sc(p, p)

    @pl.loop(_U, NJD, step=_U)
    def _(j):
        for p in range(_U):
            wait_sc(p)
            start_sc(p, j + p)

    for p in range(_U):
        wait_sc(p)
    plsc.subcore_barrier()
    pltpu.sync_copy(dacc.at[pl.ds(s * RPS, RPS)], out_hbm.at[c, s])


@functools.lru_cache(maxsize=None)
def _build_sc_kernels():
    mesh = plsc.VectorSubcoreMesh(
        core_axis_name="c", subcore_axis_name="s",
        num_cores=NC, num_subcores=NS)
    prop = pl.kernel(
        _sc_prop_body,
        out_type=jax.ShapeDtypeStruct((NC, NS, RPS, D), jnp.float32),
        mesh=mesh,
        scratch_types=[
            pltpu.VMEM((_IR, CH), jnp.int32),
            pltpu.VMEM((_IR, CH), jnp.int32),
            pltpu.VMEM((_U, CH, D), jnp.float32),
            pltpu.VMEM_SHARED((N, D), jnp.float32),
        ] + [pltpu.SemaphoreType.DMA] * (_IR + 2 * _U),
    )
    deg = pl.kernel(
        _sc_deg_body,
        out_type=jax.ShapeDtypeStruct((NC, NS, RPS, D), jnp.float32),
        mesh=mesh,
        scratch_types=[
            pltpu.VMEM((NJD, CH), jnp.int32),
            pltpu.VMEM((CH, D), jnp.float32),
            pltpu.VMEM_SHARED((N, D), jnp.float32),
        ] + [pltpu.SemaphoreType.DMA] * _U,
    )
    return prop, deg


def _sc_prop(scaled, gather_idx, scatter_idx, zeros):
    return _build_sc_kernels()[0](
        scaled, gather_idx, scatter_idx, zeros).reshape(NC, N, D)


def _sc_deg(row, ones, zeros):
    return _build_sc_kernels()[1](row, ones, zeros).reshape(NC, N, D)


_B = 1000


def _mlp_body(x_ref, w1_ref, c1_ref, d1_ref, w2_ref, c2_ref, d2_ref, o_ref):
    x = x_ref[...]
    h = jnp.dot(x, w1_ref[...], preferred_element_type=jnp.float32)
    h = jnp.maximum(h * c1_ref[...] + d1_ref[...], 0.0)
    h2 = jnp.dot(h, w2_ref[...], preferred_element_type=jnp.float32)
    h2 = jnp.maximum(h2 * c2_ref[...] + d2_ref[...], 0.0)
    o_ref[...] = h2 + h


def _ew0_body(d0_ref, d1_ref, base_ref, invd_ref, scaled_ref):
    deg = d0_ref[...] + d1_ref[...]
    iv = 1.0 / jnp.maximum(deg[:, 0:1], 1.0)
    invd_ref[...] = iv
    scaled_ref[...] = base_ref[...] * iv


def _ew_body(p0_ref, p1_ref, base_ref, invd_ref, xs_ref, scaled_ref):
    cur = (1.0 - ALPHA) * (p0_ref[...] + p1_ref[...]) + ALPHA * base_ref[...]
    xs_ref[...] = cur
    scaled_ref[...] = cur * invd_ref[...]


def _final_body(*refs):
    (x0_ref, x1_ref, x2_ref, x3_ref, x4_ref, x5_ref, x6_ref, x7_ref, x8_ref,
     x9_ref, x10_ref, wa1_ref, ba1_ref, wa2_ref, ba2_ref, wh1_ref, dh1_ref,
     ch_ref, wh2_ref, bh2_ref, o_ref) = refs
    xs = [x0_ref[...], x1_ref[...], x2_ref[...], x3_ref[...], x4_ref[...],
          x5_ref[...], x6_ref[...], x7_ref[...], x8_ref[...], x9_ref[...],
          x10_ref[...]]
    ctx = jnp.concatenate([xs[0], xs[K]], axis=1)
    a = jnp.dot(ctx, wa1_ref[...], preferred_element_type=jnp.float32)
    a = a + ba1_ref[...]
    a = 0.5 * a * (1.0 + lax.erf(a * _SQRT_HALF))
    logits = jnp.dot(a, wa2_ref[...], preferred_element_type=jnp.float32)
    logits = logits + ba2_ref[...]
    logits = logits - jnp.max(logits, axis=1, keepdims=True)
    ex = jnp.exp(logits)
    w = ex / jnp.sum(ex, axis=1, keepdims=True)
    fused = w[:, 0:1] * xs[0]
    for k in range(1, K + 1):
        fused = fused + w[:, k:k + 1] * xs[k]
    o1 = jnp.dot(fused, wh1_ref[...], preferred_element_type=jnp.float32)
    o1 = jnp.maximum(o1 * ch_ref[...] + dh1_ref[...], 0.0)
    out = jnp.dot(o1, wh2_ref[...], preferred_element_type=jnp.float32)
    o_ref[...] = out + bh2_ref[...]


def _row_spec(width):
    return pl.BlockSpec((_B, width), lambda i: (i, 0))


def _full_spec(shape):
    return pl.BlockSpec(shape, lambda i: tuple(0 for _ in shape))


def _tc_call(body, in_specs, out_specs, out_shape):
    return pl.pallas_call(
        body,
        grid=(N // _B,),
        in_specs=in_specs,
        out_specs=out_specs,
        out_shape=out_shape,
    )


def kernel(x, edge_index, W1, b1, g1, be1, W2, b2, g2, be2, Wa1, ba1, Wa2,
           ba2, Wh1, bh1, gh, beh, Wh2, bh2):
    f32 = jnp.float32
    row = edge_index[0].reshape(NW, NJ, CH)
    col = edge_index[1].reshape(NW, NJ, CH)
    zeros128 = jnp.zeros((RPS, D), f32)
    ones128 = jnp.ones((CH, D), f32)

    c1 = (g1 * _BN_INV)[None, :]
    d1 = (b1 * g1 * _BN_INV + be1)[None, :]
    c2 = (g2 * _BN_INV)[None, :]
    d2 = (b2 * g2 * _BN_INV + be2)[None, :]
    ch = (gh * _BN_INV)[None, :]
    dh1 = (bh1 * gh * _BN_INV + beh)[None, :]

    base = _tc_call(
        _mlp_body,
        [_row_spec(D), _full_spec((D, D)), _full_spec((1, D)),
         _full_spec((1, D)), _full_spec((D, D)), _full_spec((1, D)),
         _full_spec((1, D))],
        _row_spec(D),
        jax.ShapeDtypeStruct((N, D), f32),
    )(x, W1.T, c1, d1, W2.T, c2, d2)

    degp = _sc_deg(row, ones128, zeros128)

    invd, scaled = _tc_call(
        _ew0_body,
        [_row_spec(D), _row_spec(D), _row_spec(D)],
        [_row_spec(1), _row_spec(D)],
        [jax.ShapeDtypeStruct((N, 1), f32), jax.ShapeDtypeStruct((N, D), f32)],
    )(degp[0], degp[1], base)

    xs = [base]
    for _ in range(K):
        parts = _sc_prop(scaled, row, col, zeros128)
        xt, scaled = _tc_call(
            _ew_body,
            [_row_spec(D), _row_spec(D), _row_spec(D), _row_spec(1)],
            [_row_spec(D), _row_spec(D)],
            [jax.ShapeDtypeStruct((N, D), f32),
             jax.ShapeDtypeStruct((N, D), f32)],
        )(parts[0], parts[1], base, invd)
        xs.append(xt)

    att_hidden = Wa1.shape[0]
    out = _tc_call(
        _final_body,
        [_row_spec(D)] * (K + 1) + [
            _full_spec((2 * D, att_hidden)), _full_spec((1, att_hidden)),
            _full_spec((att_hidden, K + 1)), _full_spec((1, K + 1)),
            _full_spec((D, D // 2)), _full_spec((1, D // 2)),
            _full_spec((1, D // 2)), _full_spec((D // 2, D)),
            _full_spec((1, D)),
        ],
        _row_spec(D),
        jax.ShapeDtypeStruct((N, D), f32),
    )(*xs, Wa1.T, ba1[None, :], Wa2.T, ba2[None, :], Wh1.T, dh1, ch, Wh2.T,
      bh2[None, :])
    return out


# speedup vs baseline: 1.2516x; 1.0849x over previous
import functools

import jax
import jax.numpy as jnp
from jax import lax
from jax.experimental import pallas as pl
from jax.experimental.pallas import tpu as pltpu
from jax.experimental.pallas import tpu_sc as plsc

N = 10000
D = 128
E = 320000
K = 10
ALPHA = 0.1

NC = 2
NS = 16
NW = NC * NS
CH = 40
NJ = E // NW // CH
RPS = N // NS

_BN_INV = 1.0 / (1.0 + 1e-5) ** 0.5
_SQRT_HALF = 0.7071067811865476

_U = 5
_IR = 10
_GD = 4


def _sc_prop_body(scaled_hbm, row_hbm, col_hbm, zeros_hbm, out_hbm,
                  irow, icol, gb, acc, *sems):
    isems, gsems, ssems = sems[:_IR], sems[_IR:_IR + _U], sems[_IR + _U:]
    c = lax.axis_index("c")
    s = lax.axis_index("s")
    wid = c * NS + s

    def load_idx(slot, j):
        pltpu.async_copy(row_hbm.at[wid, j], irow.at[slot], isems[slot])
        pltpu.async_copy(col_hbm.at[wid, j], icol.at[slot], isems[slot])

    def wait_idx(slot, j):
        pltpu.make_async_copy(
            row_hbm.at[wid, j], irow.at[slot], isems[slot]).wait()
        pltpu.make_async_copy(
            col_hbm.at[wid, j], icol.at[slot], isems[slot]).wait()

    def start_gather(gslot, islot):
        pltpu.async_copy(scaled_hbm.at[irow.at[islot]], gb.at[gslot],
                         gsems[gslot])

    def wait_gather(gslot, islot):
        pltpu.make_async_copy(scaled_hbm.at[irow.at[islot]], gb.at[gslot],
                              gsems[gslot]).wait()

    def start_scatter(gslot, islot):
        pltpu.async_copy(gb.at[gslot], acc.at[icol.at[islot]], ssems[gslot],
                         add=True)

    def wait_scatter(gslot, islot):
        pltpu.make_async_copy(gb.at[gslot], acc.at[icol.at[islot]],
                              ssems[gslot]).wait()

    pltpu.sync_copy(zeros_hbm, acc.at[pl.ds(s * RPS, RPS)])

    for p in range(_IR):
        load_idx(p, p)
    plsc.subcore_barrier()
    for p in range(_GD):
        wait_idx(p, p)
        start_gather(p, p)

    @pl.loop(0, NJ, step=_IR)
    def _(j):
        for p in range(_IR):
            jj = j + p
            q = p % _U
            qm, pm = (q - 1) % _U, (p - 1) % _IR
            qg, pg = (q + _GD) % _U, (p + _GD) % _IR

            @pl.when(jj >= 1)
            def _():
                wait_scatter(qm, pm)

                @pl.when(jj - 1 + _IR < NJ)
                def _():
                    load_idx(pm, jj - 1 + _IR)

            @pl.when(jj + _GD < NJ)
            def _():
                wait_idx(pg, jj + _GD)
                start_gather(qg, pg)

            wait_gather(q, p)
            start_scatter(q, p)

    wait_scatter((NJ - 1) % _U, (NJ - 1) % _IR)
    plsc.subcore_barrier()
    pltpu.sync_copy(acc.at[pl.ds(s * RPS, RPS)], out_hbm.at[c, s])


NJD = E // NW // CH


def _sc_deg_body(row_hbm, ones_hbm, zeros_hbm, out_hbm, rowbuf, ones_v, dacc,
                 *sems):
    c = lax.axis_index("c")
    s = lax.axis_index("s")
    wid = c * NS + s

    pltpu.sync_copy(row_hbm.at[wid], rowbuf)
    pltpu.sync_copy(ones_hbm, ones_v)
    pltpu.sync_copy(zeros_hbm, dacc.at[pl.ds(s * RPS, RPS)])
    plsc.subcore_barrier()

    def start_sc(slot, j):
        pltpu.async_copy(ones_v, dacc.at[rowbuf.at[j]], sems[slot], add=True)

    def wait_sc(slot):
        pltpu.make_async_copy(ones_v, dacc.at[rowbuf.at[0]],
                              sems[slot]).wait()

    for p in range(_U):
        start_sc(p, p)

    @pl.loop(_U, NJD, step=_U)
    def _(j):
        for p in range(_U):
            wait_sc(p)
            start_sc(p, j + p)

    for p in range(_U):
        wait_sc(p)
    plsc.subcore_barrier()
    pltpu.sync_copy(dacc.at[pl.ds(s * RPS, RPS)], out_hbm.at[c, s])


@functools.lru_cache(maxsize=None)
def _build_sc_kernels():
    mesh = plsc.VectorSubcoreMesh(
        core_axis_name="c", subcore_axis_name="s",
        num_cores=NC, num_subcores=NS)
    prop = pl.kernel(
        _sc_prop_body,
        out_type=jax.ShapeDtypeStruct((NC, NS, RPS, D), jnp.float32),
        mesh=mesh,
        scratch_types=[
            pltpu.VMEM((_IR, CH), jnp.int32),
            pltpu.VMEM((_IR, CH), jnp.int32),
            pltpu.VMEM((_U, CH, D), jnp.float32),
            pltpu.VMEM_SHARED((N, D), jnp.float32),
        ] + [pltpu.SemaphoreType.DMA] * (_IR + 2 * _U),
    )
    deg = pl.kernel(
        _sc_deg_body,
        out_type=jax.ShapeDtypeStruct((NC, NS, RPS, D), jnp.float32),
        mesh=mesh,
        scratch_types=[
            pltpu.VMEM((NJD, CH), jnp.int32),
            pltpu.VMEM((CH, D), jnp.float32),
            pltpu.VMEM_SHARED((N, D), jnp.float32),
        ] + [pltpu.SemaphoreType.DMA] * _U,
    )
    return prop, deg


def _sc_prop(scaled, gather_idx, scatter_idx, zeros):
    return _build_sc_kernels()[0](scaled, gather_idx, scatter_idx, zeros)


def _sc_deg(row, ones, zeros):
    return _build_sc_kernels()[1](row, ones, zeros)


_B = 1000


def _mlp_body(x_ref, w1_ref, c1_ref, d1_ref, w2_ref, c2_ref, d2_ref, o_ref):
    x = x_ref[...]
    h = jnp.dot(x, w1_ref[...], preferred_element_type=jnp.float32)
    h = jnp.maximum(h * c1_ref[...] + d1_ref[...], 0.0)
    h2 = jnp.dot(h, w2_ref[...], preferred_element_type=jnp.float32)
    h2 = jnp.maximum(h2 * c2_ref[...] + d2_ref[...], 0.0)
    o_ref[...] = h2 + h


_BG = 8
_BR = _BG * RPS


def _ew0_body(d0_ref, d1_ref, base_ref, invd_ref, scaled_ref):
    deg = (d0_ref[0] + d1_ref[0]).reshape(_BR, D)
    iv = 1.0 / jnp.maximum(deg[:, 0:1], 1.0)
    invd_ref[...] = iv
    scaled_ref[...] = base_ref[...] * iv


def _ew_body(p0_ref, p1_ref, base_ref, invd_ref, xs_ref, scaled_ref):
    agg = (p0_ref[0] + p1_ref[0]).reshape(_BR, D)
    cur = (1.0 - ALPHA) * agg + ALPHA * base_ref[...]
    xs_ref[...] = cur
    scaled_ref[...] = cur * invd_ref[...]


def _final_body(*refs):
    (x0_ref, x1_ref, x2_ref, x3_ref, x4_ref, x5_ref, x6_ref, x7_ref, x8_ref,
     x9_ref, x10_ref, wa1_ref, ba1_ref, wa2_ref, ba2_ref, wh1_ref, dh1_ref,
     ch_ref, wh2_ref, bh2_ref, o_ref) = refs
    xs = [x0_ref[...], x1_ref[...], x2_ref[...], x3_ref[...], x4_ref[...],
          x5_ref[...], x6_ref[...], x7_ref[...], x8_ref[...], x9_ref[...],
          x10_ref[...]]
    ctx = jnp.concatenate([xs[0], xs[K]], axis=1)
    a = jnp.dot(ctx, wa1_ref[...], preferred_element_type=jnp.float32)
    a = a + ba1_ref[...]
    a = 0.5 * a * (1.0 + lax.erf(a * _SQRT_HALF))
    logits = jnp.dot(a, wa2_ref[...], preferred_element_type=jnp.float32)
    logits = logits + ba2_ref[...]
    logits = logits - jnp.max(logits, axis=1, keepdims=True)
    ex = jnp.exp(logits)
    w = ex / jnp.sum(ex, axis=1, keepdims=True)
    fused = w[:, 0:1] * xs[0]
    for k in range(1, K + 1):
        fused = fused + w[:, k:k + 1] * xs[k]
    o1 = jnp.dot(fused, wh1_ref[...], preferred_element_type=jnp.float32)
    o1 = jnp.maximum(o1 * ch_ref[...] + dh1_ref[...], 0.0)
    out = jnp.dot(o1, wh2_ref[...], preferred_element_type=jnp.float32)
    o_ref[...] = out + bh2_ref[...]


def _row_spec(width):
    return pl.BlockSpec((_B, width), lambda i: (i, 0))


def _part_spec(core):
    return pl.BlockSpec((1, _BG, RPS, D), lambda i: (core, i, 0, 0))


def _ewrow_spec(width):
    return pl.BlockSpec((_BR, width), lambda i: (i, 0))


def _ew_call(body, in_specs, out_specs, out_shape):
    return pl.pallas_call(
        body,
        grid=(NS // _BG,),
        in_specs=in_specs,
        out_specs=out_specs,
        out_shape=out_shape,
    )


def _full_spec(shape):
    return pl.BlockSpec(shape, lambda i: tuple(0 for _ in shape))


def _tc_call(body, in_specs, out_specs, out_shape):
    return pl.pallas_call(
        body,
        grid=(N // _B,),
        in_specs=in_specs,
        out_specs=out_specs,
        out_shape=out_shape,
    )


def kernel(x, edge_index, W1, b1, g1, be1, W2, b2, g2, be2, Wa1, ba1, Wa2,
           ba2, Wh1, bh1, gh, beh, Wh2, bh2):
    f32 = jnp.float32
    row = edge_index[0].reshape(NW, NJ, CH)
    col = edge_index[1].reshape(NW, NJ, CH)
    zeros128 = jnp.zeros((RPS, D), f32)
    ones128 = jnp.ones((CH, D), f32)

    c1 = (g1 * _BN_INV)[None, :]
    d1 = (b1 * g1 * _BN_INV + be1)[None, :]
    c2 = (g2 * _BN_INV)[None, :]
    d2 = (b2 * g2 * _BN_INV + be2)[None, :]
    ch = (gh * _BN_INV)[None, :]
    dh1 = (bh1 * gh * _BN_INV + beh)[None, :]

    base = _tc_call(
        _mlp_body,
        [_row_spec(D), _full_spec((D, D)), _full_spec((1, D)),
         _full_spec((1, D)), _full_spec((D, D)), _full_spec((1, D)),
         _full_spec((1, D))],
        _row_spec(D),
        jax.ShapeDtypeStruct((N, D), f32),
    )(x, W1.T, c1, d1, W2.T, c2, d2)

    degp = _sc_deg(row, ones128, zeros128)

    invd, scaled = _ew_call(
        _ew0_body,
        [_part_spec(0), _part_spec(1), _ewrow_spec(D)],
        [_ewrow_spec(1), _ewrow_spec(D)],
        [jax.ShapeDtypeStruct((N, 1), f32), jax.ShapeDtypeStruct((N, D), f32)],
    )(degp, degp, base)

    xs = [base]
    for _ in range(K):
        parts = _sc_prop(scaled, row, col, zeros128)
        xt, scaled = _ew_call(
            _ew_body,
            [_part_spec(0), _part_spec(1), _ewrow_spec(D), _ewrow_spec(1)],
            [_ewrow_spec(D), _ewrow_spec(D)],
            [jax.ShapeDtypeStruct((N, D), f32),
             jax.ShapeDtypeStruct((N, D), f32)],
        )(parts, parts, base, invd)
        xs.append(xt)

    att_hidden = Wa1.shape[0]
    out = _tc_call(
        _final_body,
        [_row_spec(D)] * (K + 1) + [
            _full_spec((2 * D, att_hidden)), _full_spec((1, att_hidden)),
            _full_spec((att_hidden, K + 1)), _full_spec((1, K + 1)),
            _full_spec((D, D // 2)), _full_spec((1, D // 2)),
            _full_spec((1, D // 2)), _full_spec((D // 2, D)),
            _full_spec((1, D)),
        ],
        _row_spec(D),
        jax.ShapeDtypeStruct((N, D), f32),
    )(*xs, Wa1.T, ba1[None, :], Wa2.T, ba2[None, :], Wh1.T, dh1, ch, Wh2.T,
      bh2[None, :])
    return out


# speedup vs baseline: 1.3230x; 1.0570x over previous
import functools

import jax
import jax.numpy as jnp
from jax import lax
from jax.experimental import pallas as pl
from jax.experimental.pallas import tpu as pltpu
from jax.experimental.pallas import tpu_sc as plsc

N = 10000
D = 128
E = 320000
K = 10
ALPHA = 0.1

NC = 2
NS = 16
NW = NC * NS
CH = 40
NJ = E // NW // CH
DSZ = 624
DTL = N - DSZ * NS

_BN_INV = 1.0 / (1.0 + 1e-5) ** 0.5
_SQRT_HALF = 0.7071067811865476

_U = 5
_IR = 10
_GD = 4


def _sc_prop_body(scaled_hbm, row_hbm, col_hbm, zeros_hbm, out_hbm,
                  irow, icol, gb, acc, *sems):
    isems, gsems, ssems = sems[:_IR], sems[_IR:_IR + _U], sems[_IR + _U:]
    c = lax.axis_index("c")
    s = lax.axis_index("s")
    wid = c * NS + s

    def load_idx(slot, j):
        pltpu.async_copy(row_hbm.at[wid, j], irow.at[slot], isems[slot])
        pltpu.async_copy(col_hbm.at[wid, j], icol.at[slot], isems[slot])

    def wait_idx(slot, j):
        pltpu.make_async_copy(
            row_hbm.at[wid, j], irow.at[slot], isems[slot]).wait()
        pltpu.make_async_copy(
            col_hbm.at[wid, j], icol.at[slot], isems[slot]).wait()

    def start_gather(gslot, islot):
        pltpu.async_copy(scaled_hbm.at[irow.at[islot]], gb.at[gslot],
                         gsems[gslot])

    def wait_gather(gslot, islot):
        pltpu.make_async_copy(scaled_hbm.at[irow.at[islot]], gb.at[gslot],
                              gsems[gslot]).wait()

    def start_scatter(gslot, islot):
        pltpu.async_copy(gb.at[gslot], acc.at[icol.at[islot]], ssems[gslot],
                         add=True)

    def wait_scatter(gslot, islot):
        pltpu.make_async_copy(gb.at[gslot], acc.at[icol.at[islot]],
                              ssems[gslot]).wait()

    pltpu.sync_copy(zeros_hbm.at[pl.ds(0, DSZ)], acc.at[pl.ds(s * DSZ, DSZ)])

    @pl.when(s == NS - 1)
    def _():
        pltpu.sync_copy(zeros_hbm.at[pl.ds(0, DTL)],
                        acc.at[pl.ds(DSZ * NS, DTL)])

    for p in range(_IR):
        load_idx(p, p)
    plsc.subcore_barrier()
    for p in range(_GD):
        wait_idx(p, p)
        start_gather(p, p)

    @pl.loop(0, NJ, step=_IR)
    def _(j):
        for p in range(_IR):
            jj = j + p
            q = p % _U
            qm, pm = (q - 1) % _U, (p - 1) % _IR
            qg, pg = (q + _GD) % _U, (p + _GD) % _IR

            @pl.when(jj >= 1)
            def _():
                wait_scatter(qm, pm)

                @pl.when(jj - 1 + _IR < NJ)
                def _():
                    load_idx(pm, jj - 1 + _IR)

            @pl.when(jj + _GD < NJ)
            def _():
                wait_idx(pg, jj + _GD)
                start_gather(qg, pg)

            wait_gather(q, p)
            start_scatter(q, p)

    wait_scatter((NJ - 1) % _U, (NJ - 1) % _IR)
    plsc.subcore_barrier()
    pltpu.sync_copy(acc.at[pl.ds(s * DSZ, DSZ)],
                    out_hbm.at[c, pl.ds(s * DSZ, DSZ)])

    @pl.when(s == NS - 1)
    def _():
        pltpu.sync_copy(acc.at[pl.ds(DSZ * NS, DTL)],
                        out_hbm.at[c, pl.ds(DSZ * NS, DTL)])


NJD = E // NW // CH


def _sc_deg_body(row_hbm, ones_hbm, zeros_hbm, out_hbm, rowbuf, ones_v, dacc,
                 *sems):
    c = lax.axis_index("c")
    s = lax.axis_index("s")
    wid = c * NS + s

    pltpu.sync_copy(row_hbm.at[wid], rowbuf)
    pltpu.sync_copy(ones_hbm, ones_v)
    pltpu.sync_copy(zeros_hbm.at[pl.ds(0, DSZ)], dacc.at[pl.ds(s * DSZ, DSZ)])

    @pl.when(s == NS - 1)
    def _():
        pltpu.sync_copy(zeros_hbm.at[pl.ds(0, DTL)],
                        dacc.at[pl.ds(DSZ * NS, DTL)])

    plsc.subcore_barrier()

    def start_sc(slot, j):
        pltpu.async_copy(ones_v, dacc.at[rowbuf.at[j]], sems[slot], add=True)

    def wait_sc(slot):
        pltpu.make_async_copy(ones_v, dacc.at[rowbuf.at[0]],
                              sems[slot]).wait()

    for p in range(_U):
        start_sc(p, p)

    @pl.loop(_U, NJD, step=_U)
    def _(j):
        for p in range(_U):
            wait_sc(p)
            start_sc(p, j + p)

    for p in range(_U):
        wait_sc(p)
    plsc.subcore_barrier()
    pltpu.sync_copy(dacc.at[pl.ds(s * DSZ, DSZ)],
                    out_hbm.at[c, pl.ds(s * DSZ, DSZ)])

    @pl.when(s == NS - 1)
    def _():
        pltpu.sync_copy(dacc.at[pl.ds(DSZ * NS, DTL)],
                        out_hbm.at[c, pl.ds(DSZ * NS, DTL)])


@functools.lru_cache(maxsize=None)
def _build_sc_kernels():
    mesh = plsc.VectorSubcoreMesh(
        core_axis_name="c", subcore_axis_name="s",
        num_cores=NC, num_subcores=NS)
    prop = pl.kernel(
        _sc_prop_body,
        out_type=jax.ShapeDtypeStruct((NC, N, D), jnp.float32),
        mesh=mesh,
        scratch_types=[
            pltpu.VMEM((_IR, CH), jnp.int32),
            pltpu.VMEM((_IR, CH), jnp.int32),
            pltpu.VMEM((_U, CH, D), jnp.float32),
            pltpu.VMEM_SHARED((N, D), jnp.float32),
        ] + [pltpu.SemaphoreType.DMA] * (_IR + 2 * _U),
    )
    deg = pl.kernel(
        _sc_deg_body,
        out_type=jax.ShapeDtypeStruct((NC, N, D), jnp.float32),
        mesh=mesh,
        scratch_types=[
            pltpu.VMEM((NJD, CH), jnp.int32),
            pltpu.VMEM((CH, D), jnp.float32),
            pltpu.VMEM_SHARED((N, D), jnp.float32),
        ] + [pltpu.SemaphoreType.DMA] * _U,
    )
    return prop, deg


def _sc_prop(scaled, gather_idx, scatter_idx, zeros):
    return _build_sc_kernels()[0](scaled, gather_idx, scatter_idx, zeros)


def _sc_deg(row, ones, zeros):
    return _build_sc_kernels()[1](row, ones, zeros)


_B = 1000


def _mlp_body(x_ref, w1_ref, c1_ref, d1_ref, w2_ref, c2_ref, d2_ref, o_ref):
    x = x_ref[...]
    h = jnp.dot(x, w1_ref[...], preferred_element_type=jnp.float32)
    h = jnp.maximum(h * c1_ref[...] + d1_ref[...], 0.0)
    h2 = jnp.dot(h, w2_ref[...], preferred_element_type=jnp.float32)
    h2 = jnp.maximum(h2 * c2_ref[...] + d2_ref[...], 0.0)
    o_ref[...] = h2 + h


def _ew0_body(d0_ref, d1_ref, base_ref, invd_ref, scaled_ref):
    deg = d0_ref[0] + d1_ref[0]
    iv = 1.0 / jnp.maximum(deg[:, 0:1], 1.0)
    invd_ref[...] = iv
    scaled_ref[...] = base_ref[...] * iv


def _ew_body(p0_ref, p1_ref, base_ref, invd_ref, xs_ref, scaled_ref):
    cur = ((1.0 - ALPHA) * (p0_ref[0] + p1_ref[0])
           + ALPHA * base_ref[...])
    xs_ref[...] = cur
    scaled_ref[...] = cur * invd_ref[...]


def _final_body(*refs):
    (x0_ref, x1_ref, x2_ref, x3_ref, x4_ref, x5_ref, x6_ref, x7_ref, x8_ref,
     x9_ref, x10_ref, wa1_ref, ba1_ref, wa2_ref, ba2_ref, wh1_ref, dh1_ref,
     ch_ref, wh2_ref, bh2_ref, o_ref) = refs
    xs = [x0_ref[...], x1_ref[...], x2_ref[...], x3_ref[...], x4_ref[...],
          x5_ref[...], x6_ref[...], x7_ref[...], x8_ref[...], x9_ref[...],
          x10_ref[...]]
    ctx = jnp.concatenate([xs[0], xs[K]], axis=1)
    a = jnp.dot(ctx, wa1_ref[...], preferred_element_type=jnp.float32)
    a = a + ba1_ref[...]
    a = 0.5 * a * (1.0 + lax.erf(a * _SQRT_HALF))
    logits = jnp.dot(a, wa2_ref[...], preferred_element_type=jnp.float32)
    logits = logits + ba2_ref[...]
    logits = logits - jnp.max(logits, axis=1, keepdims=True)
    ex = jnp.exp(logits)
    w = ex / jnp.sum(ex, axis=1, keepdims=True)
    fused = w[:, 0:1] * xs[0]
    for k in range(1, K + 1):
        fused = fused + w[:, k:k + 1] * xs[k]
    o1 = jnp.dot(fused, wh1_ref[...], preferred_element_type=jnp.float32)
    o1 = jnp.maximum(o1 * ch_ref[...] + dh1_ref[...], 0.0)
    out = jnp.dot(o1, wh2_ref[...], preferred_element_type=jnp.float32)
    o_ref[...] = out + bh2_ref[...]


def _row_spec(width):
    return pl.BlockSpec((_B, width), lambda i: (i, 0))


def _part_spec(core):
    return pl.BlockSpec((1, _B, D), lambda i: (core, i, 0))


def _full_spec(shape):
    return pl.BlockSpec(shape, lambda i: tuple(0 for _ in shape))


def _tc_call(body, in_specs, out_specs, out_shape):
    return pl.pallas_call(
        body,
        grid=(N // _B,),
        in_specs=in_specs,
        out_specs=out_specs,
        out_shape=out_shape,
    )


def kernel(x, edge_index, W1, b1, g1, be1, W2, b2, g2, be2, Wa1, ba1, Wa2,
           ba2, Wh1, bh1, gh, beh, Wh2, bh2):
    f32 = jnp.float32
    row = edge_index[0].reshape(NW, NJ, CH)
    col = edge_index[1].reshape(NW, NJ, CH)
    zeros128 = jnp.zeros((DSZ, D), f32)
    ones128 = jnp.ones((CH, D), f32)

    c1 = (g1 * _BN_INV)[None, :]
    d1 = (b1 * g1 * _BN_INV + be1)[None, :]
    c2 = (g2 * _BN_INV)[None, :]
    d2 = (b2 * g2 * _BN_INV + be2)[None, :]
    ch = (gh * _BN_INV)[None, :]
    dh1 = (bh1 * gh * _BN_INV + beh)[None, :]

    base = _tc_call(
        _mlp_body,
        [_row_spec(D), _full_spec((D, D)), _full_spec((1, D)),
         _full_spec((1, D)), _full_spec((D, D)), _full_spec((1, D)),
         _full_spec((1, D))],
        _row_spec(D),
        jax.ShapeDtypeStruct((N, D), f32),
    )(x, W1.T, c1, d1, W2.T, c2, d2)

    degp = _sc_deg(row, ones128, zeros128)

    invd, scaled = _tc_call(
        _ew0_body,
        [_part_spec(0), _part_spec(1), _row_spec(D)],
        [_row_spec(1), _row_spec(D)],
        [jax.ShapeDtypeStruct((N, 1), f32), jax.ShapeDtypeStruct((N, D), f32)],
    )(degp, degp, base)

    xs = [base]
    for _ in range(K):
        parts = _sc_prop(scaled, row, col, zeros128)
        xt, scaled = _tc_call(
            _ew_body,
            [_part_spec(0), _part_spec(1), _row_spec(D), _row_spec(1)],
            [_row_spec(D), _row_spec(D)],
            [jax.ShapeDtypeStruct((N, D), f32),
             jax.ShapeDtypeStruct((N, D), f32)],
        )(parts, parts, base, invd)
        xs.append(xt)

    att_hidden = Wa1.shape[0]
    out = _tc_call(
        _final_body,
        [_row_spec(D)] * (K + 1) + [
            _full_spec((2 * D, att_hidden)), _full_spec((1, att_hidden)),
            _full_spec((att_hidden, K + 1)), _full_spec((1, K + 1)),
            _full_spec((D, D // 2)), _full_spec((1, D // 2)),
            _full_spec((1, D // 2)), _full_spec((D // 2, D)),
            _full_spec((1, D)),
        ],
        _row_spec(D),
        jax.ShapeDtypeStruct((N, D), f32),
    )(*xs, Wa1.T, ba1[None, :], Wa2.T, ba2[None, :], Wh1.T, dh1, ch, Wh2.T,
      bh2[None, :])
    return out
